# baseline (speedup 1.0000x reference)
import functools

import jax
import jax.numpy as jnp
from jax import lax
from jax.experimental import pallas as pl
from jax.experimental.pallas import tpu as pltpu
from jax.experimental.pallas import tpu_sc as plsc

NUM_GRAPHS = 128
N_NODES = 100000
N_EDGES = 1600000
D_NODE = 128
D_EDGE = 16
D_GLOBAL = 64
D_OUT = 128

CH = 128
BLK = 8
NW = 32

N_CHUNKS_N = N_NODES // CH
N_TAIL = N_NODES - N_CHUNKS_N * CH
NBLK_N = N_CHUNKS_N // BLK
NTAIL_CH_N = N_CHUNKS_N - NBLK_N * BLK
N_IDROWS_N = (NBLK_N + 1) * BLK

N_CHUNKS_E = N_EDGES // CH
NBLK_E = N_CHUNKS_E // BLK
NTAIL_CH_E = N_CHUNKS_E - NBLK_E * BLK
N_IDROWS_E = (NBLK_E + 1) * BLK

NB_N, EX_N = NBLK_N // NW, NBLK_N % NW
NB_E, EX_E = NBLK_E // NW, NBLK_E % NW

HCH = 32
EGROUPS = BLK * CH // 16


def _sc_body(nfeat, eT, nids2d, ntail_ids, eids2d,
             out_n, out_e3,
             eids_v, efeat_v, nids_all, nfeat_v, ntail_ids_v,
             acc_n, acc_e3, se0, se1, se2, sn0, sn1, sn2):
    cc = lax.axis_index("c")
    sid = lax.axis_index("s")
    w = sid * 2 + cc
    zvec = jnp.zeros((16,), jnp.float32)
    lane = lax.broadcasted_iota(jnp.int32, (16,), 0)

    def zrow(i, _):
        for k in range(D_NODE // 16):
            acc_n[i, pl.ds(k * 16, 16)] = zvec
        for k in range(16):
            acc_e3[i, pl.ds(k * 16, 16)] = zvec
        return 0

    lax.fori_loop(0, NUM_GRAPHS, zrow, 0)

    eb0 = w * NB_E + jnp.minimum(w, EX_E)
    ebcnt = NB_E + jnp.where(w < EX_E, 1, 0)

    def e_flush(cur, accs):
        @pl.when(cur >= 0)
        def _():
            for r in range(16):
                acc_e3[cur, pl.ds(r * 16, 16)] += accs[r]

    def e_stage(b, slot, sem):
        chunk0 = (eb0 + b) * BLK
        pltpu.async_copy(eids2d.at[pl.ds(chunk0, BLK)], eids_v.at[slot], sem)
        pltpu.async_copy(eT.at[:, pl.ds(chunk0 * CH, BLK * CH)],
                         efeat_v.at[slot], sem)

    def e_wait(b, slot, sem):
        chunk0 = (eb0 + b) * BLK
        pltpu.make_async_copy(eids2d.at[pl.ds(chunk0, BLK)],
                              eids_v.at[slot], sem).wait()
        pltpu.make_async_copy(eT.at[:, pl.ds(chunk0 * CH, BLK * CH)],
                              efeat_v.at[slot], sem).wait()

    def e_block(slot, nchunks):
        bid0 = eids_v[slot, 0, pl.ds(0, 16)][0]
        bidL = eids_v[slot, nchunks - 1, pl.ds(CH - 16, 16)][15]

        @pl.when(bid0 == bidL)
        def _():
            def g4(i, accs):
                for u in range(4):
                    accs = tuple(
                        accs[r] + efeat_v[slot, r, pl.ds(i * 64 + u * 16, 16)]
                        for r in range(16))
                return accs

            accs = lax.fori_loop(0, nchunks * 2, g4, (zvec,) * 16)
            for r in range(16):
                acc_e3[bid0, pl.ds(r * 16, 16)] += accs[r]

        @pl.when(bid0 != bidL)
        def _():
            def grp(g, c):
                cur, accs = c[0], c[1:]
                j = g // 8
                col = (g - j * 8) * 16
                idvec = eids_v[slot, j, pl.ds(col, 16)]
                gid0 = idvec[0]
                gidL = idvec[15]
                gcol = g * 16
                uniform = gid0 == gidL
                is_new = jnp.logical_and(uniform, gid0 != cur)
                vs = [efeat_v[slot, r, pl.ds(gcol, 16)] for r in range(16)]

                @pl.when(jnp.logical_and(is_new, cur >= 0))
                def _():
                    for r in range(16):
                        acc_e3[cur, pl.ds(r * 16, 16)] += accs[r]

                @pl.when(jnp.logical_not(uniform))
                def _():
                    for r in range(16):
                        plsc.addupdate_scatter(acc_e3,
                                               [idvec, lane + r * 16], vs[r])

                accs = tuple(
                    jnp.where(uniform,
                              jnp.where(is_new, zvec, accs[r]) + vs[r],
                              accs[r])
                    for r in range(16))
                cur = jnp.where(uniform, gid0, cur)
                return (cur,) + accs

            c = lax.fori_loop(0, nchunks * 8, grp,
                              (jnp.int32(-1),) + (zvec,) * 16)
            e_flush(c[0], c[1:])

    @pl.when(w == NW - 1)
    def _():
        chunk0 = NBLK_E * BLK
        pltpu.sync_copy(eids2d.at[pl.ds(chunk0, NTAIL_CH_E)],
                        eids_v.at[0, pl.ds(0, NTAIL_CH_E)])
        pltpu.sync_copy(eT.at[:, pl.ds(chunk0 * CH, NTAIL_CH_E * CH)],
                        efeat_v.at[0, :, pl.ds(0, NTAIL_CH_E * CH)])
        e_block(0, NTAIL_CH_E)

    nb0 = w * NB_N
    pltpu.sync_copy(nids2d.at[pl.ds(nb0 * BLK, 32)], nids_all)

    def n_flush(cur, accs):
        @pl.when(cur >= 0)
        def _():
            for k in range(8):
                acc_n[cur, pl.ds(k * 16, 16)] += accs[k]

    def n_stage(h, slot, sem):
        row0 = nb0 * BLK * CH + h * HCH
        pltpu.async_copy(nfeat.at[pl.ds(row0, HCH)], nfeat_v.at[slot], sem)

    def n_wait(h, slot, sem):
        row0 = nb0 * BLK * CH + h * HCH
        pltpu.make_async_copy(nfeat.at[pl.ds(row0, HCH)],
                              nfeat_v.at[slot], sem).wait()

    def n_half(slot, idrow, roff):
        hid0 = nids_all[idrow, pl.ds(roff, 16)][0]
        hidL = nids_all[idrow, pl.ds(roff + HCH - 16, 16)][15]

        @pl.when(hid0 == hidL)
        def _():
            def r4(i, accs):
                for u in range(4):
                    accs = tuple(accs[k] + nfeat_v[slot, 4 * i + u,
                                                   pl.ds(k * 16, 16)]
                                 for k in range(8))
                return accs

            accs = lax.fori_loop(0, HCH // 4, r4, (zvec,) * 8)
            for k in range(8):
                acc_n[hid0, pl.ds(k * 16, 16)] += accs[k]

        @pl.when(hid0 != hidL)
        def _():
            def grp(gr, c):
                cur, accs = c[0], c[1:]
                r0 = gr * 16
                nidvec = nids_all[idrow, pl.ds(roff + r0, 16)]
                gid0 = nidvec[0]
                gidL = nidvec[15]
                uniform = gid0 == gidL
                is_new = jnp.logical_and(uniform, gid0 != cur)

                @pl.when(jnp.logical_and(is_new, cur >= 0))
                def _():
                    for k in range(8):
                        acc_n[cur, pl.ds(k * 16, 16)] += accs[k]

                @pl.when(jnp.logical_not(uniform))
                def _():
                    for rr in range(16):
                        rid = jnp.broadcast_to(nidvec[rr], (16,))
                        for k in range(8):
                            plsc.addupdate_scatter(
                                acc_n, [rid, lane + k * 16],
                                nfeat_v[slot, r0 + rr, pl.ds(k * 16, 16)])

                tmp = tuple(jnp.where(is_new, zvec, accs[k])
                            for k in range(8))
                for rr in range(16):
                    tmp = tuple(tmp[k] + nfeat_v[slot, r0 + rr,
                                                 pl.ds(k * 16, 16)]
                                for k in range(8))
                accs = tuple(jnp.where(uniform, tmp[k], accs[k])
                             for k in range(8))
                cur = jnp.where(uniform, gid0, cur)
                return (cur,) + accs

            c = lax.fori_loop(0, HCH // 16, grp,
                             (jnp.int32(-1),) + (zvec,) * 8)
            n_flush(c[0], c[1:])

    NQ = NB_N * BLK * CH // HCH

    esems = (se0, se1, se2)
    nsems = (sn0, sn1, sn2)
    for u in range(3):
        e_stage(u, u, esems[u])
        n_stage(u, u, nsems[u])

    def macro(i, _):
        for u in range(3):
            b = 3 * i + u
            e_wait(b, u, esems[u])
            e_block(u, BLK)

            @pl.when(b + 3 < ebcnt)
            def _(b=b, u=u):
                e_stage(b + 3, u, esems[u])

            for q_off in (2 * u, 2 * u + 1):
                q = 6 * i + q_off
                qs = q_off % 3
                n_wait(q, qs, nsems[qs])
                n_half(qs, q // 4, (q % 4) * HCH)

                @pl.when(q + 3 < NQ)
                def _(q=q, qs=qs):
                    n_stage(q + 3, qs, nsems[qs])

        return 0

    lax.fori_loop(0, 16, macro, 0)

    @pl.when(ebcnt % 2 == 1)
    def _():
        e_wait(ebcnt - 1, 0, se0)
        e_block(0, BLK)

    @pl.when(w == NW - 3)
    def _():
        pltpu.sync_copy(nids2d.at[pl.ds((NBLK_N - 1) * BLK, BLK)],
                        nids_all.at[pl.ds(0, BLK)])

        def one(j, _):
            row0 = (NBLK_N - 1) * BLK * CH + j * HCH
            pltpu.sync_copy(nfeat.at[pl.ds(row0, HCH)], nfeat_v.at[0])
            n_half(0, j // 4, (j % 4) * HCH)
            return 0

        lax.fori_loop(0, BLK * CH // HCH, one, 0)

    @pl.when(w == NW - 2)
    def _():
        pltpu.sync_copy(nids2d.at[pl.ds(NBLK_N * BLK, BLK)],
                        nids_all.at[pl.ds(0, BLK)])

        def one(j, _):
            row0 = NBLK_N * BLK * CH + j * HCH
            pltpu.sync_copy(nfeat.at[pl.ds(row0, HCH)], nfeat_v.at[0])
            n_half(0, j // 4, (j % 4) * HCH)
            return 0

        lax.fori_loop(0, NTAIL_CH_N * CH // HCH, one, 0)

    @pl.when(w == NW - 1)
    def _():
        pltpu.sync_copy(ntail_ids, ntail_ids_v)
        pltpu.sync_copy(nfeat.at[pl.ds(N_CHUNKS_N * CH, N_TAIL)],
                        nfeat_v.at[0, pl.ds(0, N_TAIL)])
        for g in range(N_TAIL // 16):
            idvec = ntail_ids_v[pl.ds(g * 16, 16)]
            for rr in range(16):
                rid = jnp.broadcast_to(idvec[rr], (16,))
                for k in range(8):
                    plsc.addupdate_scatter(
                        acc_n, [rid, lane + k * 16],
                        nfeat_v[0, g * 16 + rr, pl.ds(k * 16, 16)])

    pltpu.sync_copy(acc_n, out_n.at[w])
    pltpu.sync_copy(acc_e3, out_e3.at[w])


@functools.partial(
    pl.kernel,
    out_type=(
        jax.ShapeDtypeStruct((NW, NUM_GRAPHS, D_NODE), jnp.float32),
        jax.ShapeDtypeStruct((NW, NUM_GRAPHS, 16 * D_EDGE), jnp.float32),
    ),
    mesh=plsc.VectorSubcoreMesh(core_axis_name="c", subcore_axis_name="s"),
    scratch_types=[
        pltpu.VMEM((3, BLK, CH), jnp.int32),
        pltpu.VMEM((3, D_EDGE, BLK * CH), jnp.float32),
        pltpu.VMEM((32, CH), jnp.int32),
        pltpu.VMEM((3, HCH, D_NODE), jnp.float32),
        pltpu.VMEM((N_TAIL,), jnp.int32),
        pltpu.VMEM((NUM_GRAPHS, D_NODE), jnp.float32),
        pltpu.VMEM((NUM_GRAPHS, 16 * D_EDGE), jnp.float32),
        pltpu.SemaphoreType.DMA,
        pltpu.SemaphoreType.DMA,
        pltpu.SemaphoreType.DMA,
        pltpu.SemaphoreType.DMA,
        pltpu.SemaphoreType.DMA,
        pltpu.SemaphoreType.DMA,
    ],
    compiler_params=pltpu.CompilerParams(needs_layout_passes=False),
)
def _sc_segsums(nfeat, eT, nids2d, ntail_ids, eids2d, out_n, out_e3,
                *scratch):
    _sc_body(nfeat, eT, nids2d, ntail_ids, eids2d, out_n, out_e3, *scratch)


def _final_body(pn_ref, pe_ref, fold_ref, g_ref, wn_ref, we_ref, wg_ref,
                b_ref, out_ref):
    agg_n = jnp.sum(pn_ref[...], axis=0)
    pe = jnp.sum(pe_ref[...], axis=0)
    agg_e = jax.lax.dot_general(
        pe, fold_ref[...], (((1,), (0,)), ((), ())),
        precision=jax.lax.Precision.HIGHEST,
        preferred_element_type=jnp.float32)
    acc = jax.lax.dot_general(
        agg_n, wn_ref[...], (((1,), (1,)), ((), ())),
        preferred_element_type=jnp.float32)
    acc += jax.lax.dot_general(
        agg_e, we_ref[...], (((1,), (1,)), ((), ())),
        preferred_element_type=jnp.float32)
    acc += jax.lax.dot_general(
        g_ref[...], wg_ref[...], (((1,), (1,)), ((), ())),
        preferred_element_type=jnp.float32)
    out_ref[...] = acc + b_ref[...]


def kernel(node_features, edge_features, global_features, node_graph_ids,
           edge_graph_ids, W_node, W_edges, W_global, bias):
    nids2d = jnp.pad(
        node_graph_ids[:N_CHUNKS_N * CH],
        (0, N_IDROWS_N * CH - N_CHUNKS_N * CH)).reshape(N_IDROWS_N, CH)
    ntail_ids = node_graph_ids[N_CHUNKS_N * CH:]
    eids2d = edge_graph_ids.reshape(N_CHUNKS_E, CH)
    part_n, part_e3 = _sc_segsums(node_features, edge_features.T, nids2d,
                                  ntail_ids, eids2d)
    fold = (jnp.arange(16 * D_EDGE)[:, None] // 16
            == jnp.arange(D_EDGE)[None, :]).astype(jnp.float32)
    return pl.pallas_call(
        _final_body,
        out_shape=jax.ShapeDtypeStruct((NUM_GRAPHS, D_OUT), jnp.float32),
    )(part_n, part_e3, fold, global_features, W_node, W_edges, W_global,
      bias.reshape(1, D_OUT))
